# baseline (speedup 1.0000x reference)
import functools

import jax
import jax.numpy as jnp
from jax import lax
from jax.experimental import pallas as pl
from jax.experimental.pallas import tpu as pltpu
from jax.experimental.pallas import tpu_sc as plsc

INPUT_DIM = 2048
DICT_SIZE = 32768
K = 64
N_TOKENS = 128

_ENC_BD = 2048
_TOPK_BR = 16
_BLK = 128
_NBLK = DICT_SIZE // _BLK

_NC = 2
_NS = 16
_NW = _NC * _NS
_RPW = N_TOKENS // _NW
_L = 16


def _enc_body(x_ref, w_ref, b_ref, out_ref):
    out_ref[...] = jax.lax.dot_general(
        x_ref[...], w_ref[...],
        (((1,), (1,)), ((), ())),
        preferred_element_type=jnp.float32,
    ) + b_ref[...][None, :]


def _topk_body(pa_ref, acts_ref, bmp_ref, thr_ref):
    v = pa_ref[...]
    bits = jax.lax.bitcast_convert_type(v, jnp.uint32)
    u = jnp.where(v >= 0.0, bits | jnp.uint32(0x80000000), ~bits)
    thr = jnp.zeros((v.shape[0], 1), jnp.uint32)
    for b in range(31, -1, -1):
        cand = thr | jnp.uint32(1 << b)
        cnt = jnp.sum((u >= cand).astype(jnp.int32), axis=1, keepdims=True)
        thr = jnp.where(cnt >= K, cand, thr)
    mask = u >= thr
    acts_ref[...] = jnp.where(mask, jnp.maximum(v, 0.0), 0.0)
    mi = mask.astype(jnp.int32).reshape(v.shape[0], _NBLK, _BLK)
    bmp_ref[...] = jnp.max(mi, axis=2)
    thr_f = jnp.where(
        thr >= jnp.uint32(0x80000000),
        jax.lax.bitcast_convert_type(thr & jnp.uint32(0x7FFFFFFF), jnp.float32),
        jax.lax.bitcast_convert_type(~thr, jnp.float32),
    )
    thr_ref[...] = jnp.broadcast_to(thr_f, (v.shape[0], _L))


def _sc_body(pab_hbm, bmp_hbm, thr_hbm, wdec_hbm, bdec_hbm, recon_hbm,
             bmv, thrv, blkids, cand, widx, wval, rowbuf, acc, bdec_v,
             sem0, sem1, sem2, sem3):
    wid = lax.axis_index("s") * _NC + lax.axis_index("c")
    pltpu.sync_copy(bdec_hbm, bdec_v)
    iota = lax.iota(jnp.int32, _L)
    zi = jnp.zeros((_L,), jnp.int32)
    sems = (sem0, sem1, sem2, sem3)
    nchunk_bm = _NBLK // _L
    ngrp = K // _L

    def per_row(r0, carry):
        r = wid * _RPW + r0
        pltpu.sync_copy(bmp_hbm.at[r], bmv)
        pltpu.sync_copy(thr_hbm.at[r], thrv)
        thr_s = jnp.broadcast_to(jnp.max(thrv[...], axis=0), (_L,))
        for j in range(K // _L):
            blkids[pl.ds(j * _L, _L)] = zi

        cnt = zi
        for c in range(nchunk_bm):
            m = bmv[pl.ds(c * _L, _L)] != 0
            pos = jnp.clip(cnt + plsc.cumsum(m.astype(jnp.int32)) - 1, 0, K - 1)
            plsc.store_scatter(blkids, [pos], c * _L + iota, mask=m)
            cnt = cnt + plsc.all_reduce_population_count(m)
        nblk = jnp.max(cnt, axis=0)

        base = jnp.broadcast_to(r * _NBLK, (_L,))
        cps = []
        for gch in range(K // _L):
            idxv = base + blkids[pl.ds(gch * _L, _L)]
            cps.append(pltpu.async_copy(
                pab_hbm.at[idxv], cand.at[pl.ds(gch * _L, _L)], sems[gch]))
        for cp in cps:
            cp.wait()

        zf = jnp.zeros((_L,), jnp.float32)
        for j in range(K // _L):
            widx[pl.ds(j * _L, _L)] = zi
            wval[pl.ds(j * _L, _L)] = zf

        def sel_block(b, wcnt):
            bch = b // _L
            bb = bch * _L
            blk_chunk = blkids[pl.ds(bb, _L)]
            blk_s = jnp.max(jnp.where(iota == b - bb, blk_chunk, 0), axis=0)
            colbase = jnp.broadcast_to(blk_s * _BLK, (_L,))
            for o in range(_BLK // _L):
                v = cand[b, pl.ds(o * _L, _L)]
                m = v >= thr_s
                pos = jnp.clip(
                    wcnt + plsc.cumsum(m.astype(jnp.int32)) - 1, 0, K - 1)
                plsc.store_scatter(widx, [pos], colbase + o * _L + iota, mask=m)
                plsc.store_scatter(wval, [pos], jnp.maximum(v, 0.0), mask=m)
                wcnt = wcnt + plsc.all_reduce_population_count(m)
            return wcnt

        lax.fori_loop(0, nblk, sel_block, zi)

        _GR = 8
        _NG = K // _GR
        _NB = 4
        _UNR = 4

        cps = {}
        for g in range(3):
            cps[g] = pltpu.async_copy(
                wdec_hbm.at[widx.at[pl.ds(g * _GR, _GR)]],
                rowbuf.at[g % _NB], sems[g % _NB])

        def init_body(i, _):
            for oo in range(_UNR):
                s = pl.ds((i * _UNR + oo) * _L, _L)
                acc[s] = bdec_v[s]
            return 0

        lax.fori_loop(0, INPUT_DIM // _L // _UNR, init_body, 0)
        for g in range(_NG):
            cps[g].wait()
            if g + 3 < _NG:
                cps[g + 3] = pltpu.async_copy(
                    wdec_hbm.at[widx.at[pl.ds((g + 3) * _GR, _GR)]],
                    rowbuf.at[(g + 3) % _NB], sems[(g + 3) % _NB])
            vchunk = wval[pl.ds((g // 2) * _L, _L)]
            half = (g % 2) * _GR
            vals = [jnp.broadcast_to(
                        jnp.sum(jnp.where(iota == half + j, vchunk, 0.0),
                                axis=0), (_L,))
                    for j in range(_GR)]

            def acc_body(i, _, g=g, vals=vals):
                for oo in range(_UNR):
                    s = pl.ds((i * _UNR + oo) * _L, _L)
                    a = acc[s]
                    for j in range(_GR):
                        a = a + vals[j] * rowbuf[g % _NB, j, s]
                    acc[s] = a
                return 0

            lax.fori_loop(0, INPUT_DIM // _L // _UNR, acc_body, 0)
        pltpu.sync_copy(acc, recon_hbm.at[r])
        return carry

    lax.fori_loop(0, _RPW, per_row, 0)


def kernel(x, W_enc, b_enc, W_dec, b_dec):
    x_cent = x - b_dec[None, :]

    pre_acts = pl.pallas_call(
        _enc_body,
        grid=(DICT_SIZE // _ENC_BD,),
        in_specs=[
            pl.BlockSpec((N_TOKENS, INPUT_DIM), lambda d: (0, 0)),
            pl.BlockSpec((_ENC_BD, INPUT_DIM), lambda d: (d, 0)),
            pl.BlockSpec((_ENC_BD,), lambda d: (d,)),
        ],
        out_specs=pl.BlockSpec((N_TOKENS, _ENC_BD), lambda d: (0, d)),
        out_shape=jax.ShapeDtypeStruct((N_TOKENS, DICT_SIZE), jnp.float32),
    )(x_cent, W_enc, b_enc)

    acts, bmp, thr = pl.pallas_call(
        _topk_body,
        grid=(N_TOKENS // _TOPK_BR,),
        in_specs=[pl.BlockSpec((_TOPK_BR, DICT_SIZE), lambda r: (r, 0))],
        out_specs=[
            pl.BlockSpec((_TOPK_BR, DICT_SIZE), lambda r: (r, 0)),
            pl.BlockSpec((_TOPK_BR, _NBLK), lambda r: (r, 0)),
            pl.BlockSpec((_TOPK_BR, _L), lambda r: (r, 0)),
        ],
        out_shape=[
            jax.ShapeDtypeStruct((N_TOKENS, DICT_SIZE), jnp.float32),
            jax.ShapeDtypeStruct((N_TOKENS, _NBLK), jnp.int32),
            jax.ShapeDtypeStruct((N_TOKENS, _L), jnp.float32),
        ],
    )(pre_acts)

    pa_blocks = pre_acts.reshape(N_TOKENS * _NBLK, _BLK)

    mesh = plsc.VectorSubcoreMesh(core_axis_name="c", subcore_axis_name="s",
                                  num_cores=_NC, num_subcores=_NS)
    recon = pl.kernel(
        _sc_body,
        out_type=jax.ShapeDtypeStruct((N_TOKENS, INPUT_DIM), jnp.float32),
        mesh=mesh,
        compiler_params=pltpu.CompilerParams(needs_layout_passes=False),
        scratch_types=[
            pltpu.VMEM((_NBLK,), jnp.int32),
            pltpu.VMEM((_L,), jnp.float32),
            pltpu.VMEM((K,), jnp.int32),
            pltpu.VMEM((K, _BLK), jnp.float32),
            pltpu.VMEM((K,), jnp.int32),
            pltpu.VMEM((K,), jnp.float32),
            pltpu.VMEM((4, 8, INPUT_DIM), jnp.float32),
            pltpu.VMEM((INPUT_DIM,), jnp.float32),
            pltpu.VMEM((INPUT_DIM,), jnp.float32),
            pltpu.SemaphoreType.DMA,
            pltpu.SemaphoreType.DMA,
            pltpu.SemaphoreType.DMA,
            pltpu.SemaphoreType.DMA,
        ],
    )(pa_blocks, bmp, thr, W_dec, b_dec)

    return (recon, acts)


# speedup vs baseline: 1.0040x; 1.0040x over previous
import functools

import jax
import jax.numpy as jnp
from jax import lax
from jax.experimental import pallas as pl
from jax.experimental.pallas import tpu as pltpu
from jax.experimental.pallas import tpu_sc as plsc

INPUT_DIM = 2048
DICT_SIZE = 32768
K = 64
N_TOKENS = 128

_ENC_BD = 2048
_TOPK_BR = 16
_BLK = 128
_NBLK = DICT_SIZE // _BLK

_NC = 2
_NS = 16
_NW = _NC * _NS
_RPW = N_TOKENS // _NW
_L = 16


def _enc_body(x_ref, w_ref, b_ref, out_ref):
    out_ref[...] = jax.lax.dot_general(
        x_ref[...], w_ref[...],
        (((1,), (1,)), ((), ())),
        preferred_element_type=jnp.float32,
    ) + b_ref[...][None, :]


def _topk_body(pa_ref, acts_ref, bmp_ref, thr_ref):
    v = pa_ref[...]
    bits = jax.lax.bitcast_convert_type(v, jnp.uint32)
    u = jnp.where(v >= 0.0, bits | jnp.uint32(0x80000000), ~bits)
    ones = jnp.ones((u.shape[1],), jnp.float32)
    thr = jnp.zeros((v.shape[0], 1), jnp.uint32)
    for b in range(31, -1, -1):
        cand = thr | jnp.uint32(1 << b)
        maskf = jnp.where(u >= cand, 1.0, 0.0)
        cnt = jax.lax.dot_general(
            maskf, ones, (((1,), (0,)), ((), ())),
            preferred_element_type=jnp.float32)[:, None]
        thr = jnp.where(cnt >= float(K), cand, thr)
    mask = u >= thr
    acts_ref[...] = jnp.where(mask, jnp.maximum(v, 0.0), 0.0)
    mi = mask.astype(jnp.int32).reshape(v.shape[0], _NBLK, _BLK)
    bmp_ref[...] = jnp.max(mi, axis=2)
    thr_f = jnp.where(
        thr >= jnp.uint32(0x80000000),
        jax.lax.bitcast_convert_type(thr & jnp.uint32(0x7FFFFFFF), jnp.float32),
        jax.lax.bitcast_convert_type(~thr, jnp.float32),
    )
    thr_ref[...] = jnp.broadcast_to(thr_f, (v.shape[0], _L))


def _sc_body(pab_hbm, bmp_hbm, thr_hbm, wdec_hbm, bdec_hbm, recon_hbm,
             bmv, thrv, blkids, cand, widx, wval, rowbuf, acc, bdec_v,
             sem0, sem1, sem2, sem3):
    wid = lax.axis_index("s") * _NC + lax.axis_index("c")
    pltpu.sync_copy(bdec_hbm, bdec_v)
    iota = lax.iota(jnp.int32, _L)
    zi = jnp.zeros((_L,), jnp.int32)
    sems = (sem0, sem1, sem2, sem3)
    nchunk_bm = _NBLK // _L
    ngrp = K // _L

    def per_row(r0, carry):
        r = wid * _RPW + r0
        pltpu.sync_copy(bmp_hbm.at[r], bmv)
        pltpu.sync_copy(thr_hbm.at[r], thrv)
        thr_s = jnp.broadcast_to(jnp.max(thrv[...], axis=0), (_L,))
        for j in range(K // _L):
            blkids[pl.ds(j * _L, _L)] = zi

        cnt = zi
        for c in range(nchunk_bm):
            m = bmv[pl.ds(c * _L, _L)] != 0
            pos = jnp.clip(cnt + plsc.cumsum(m.astype(jnp.int32)) - 1, 0, K - 1)
            plsc.store_scatter(blkids, [pos], c * _L + iota, mask=m)
            cnt = cnt + plsc.all_reduce_population_count(m)
        nblk = jnp.max(cnt, axis=0)

        base = jnp.broadcast_to(r * _NBLK, (_L,))
        cps = []
        for gch in range(K // _L):
            idxv = base + blkids[pl.ds(gch * _L, _L)]
            cps.append(pltpu.async_copy(
                pab_hbm.at[idxv], cand.at[pl.ds(gch * _L, _L)], sems[gch]))
        for cp in cps:
            cp.wait()

        zf = jnp.zeros((_L,), jnp.float32)
        for j in range(K // _L):
            widx[pl.ds(j * _L, _L)] = zi
            wval[pl.ds(j * _L, _L)] = zf

        def sel_block(b, wcnt):
            bch = b // _L
            bb = bch * _L
            blk_chunk = blkids[pl.ds(bb, _L)]
            blk_s = jnp.max(jnp.where(iota == b - bb, blk_chunk, 0), axis=0)
            colbase = jnp.broadcast_to(blk_s * _BLK, (_L,))
            for o in range(_BLK // _L):
                v = cand[b, pl.ds(o * _L, _L)]
                m = v >= thr_s
                pos = jnp.clip(
                    wcnt + plsc.cumsum(m.astype(jnp.int32)) - 1, 0, K - 1)
                plsc.store_scatter(widx, [pos], colbase + o * _L + iota, mask=m)
                plsc.store_scatter(wval, [pos], jnp.maximum(v, 0.0), mask=m)
                wcnt = wcnt + plsc.all_reduce_population_count(m)
            return wcnt

        lax.fori_loop(0, nblk, sel_block, zi)

        _GR = 8
        _NG = K // _GR
        _NB = 4
        _UNR = 4

        cps = {}
        for g in range(3):
            cps[g] = pltpu.async_copy(
                wdec_hbm.at[widx.at[pl.ds(g * _GR, _GR)]],
                rowbuf.at[g % _NB], sems[g % _NB])

        def init_body(i, _):
            for oo in range(_UNR):
                s = pl.ds((i * _UNR + oo) * _L, _L)
                acc[s] = bdec_v[s]
            return 0

        lax.fori_loop(0, INPUT_DIM // _L // _UNR, init_body, 0)
        for g in range(_NG):
            cps[g].wait()
            if g + 3 < _NG:
                cps[g + 3] = pltpu.async_copy(
                    wdec_hbm.at[widx.at[pl.ds((g + 3) * _GR, _GR)]],
                    rowbuf.at[(g + 3) % _NB], sems[(g + 3) % _NB])
            vchunk = wval[pl.ds((g // 2) * _L, _L)]
            half = (g % 2) * _GR
            vals = [jnp.broadcast_to(
                        jnp.sum(jnp.where(iota == half + j, vchunk, 0.0),
                                axis=0), (_L,))
                    for j in range(_GR)]

            def acc_body(i, _, g=g, vals=vals):
                for oo in range(_UNR):
                    s = pl.ds((i * _UNR + oo) * _L, _L)
                    a = acc[s]
                    for j in range(_GR):
                        a = a + vals[j] * rowbuf[g % _NB, j, s]
                    acc[s] = a
                return 0

            lax.fori_loop(0, INPUT_DIM // _L // _UNR, acc_body, 0)
        pltpu.sync_copy(acc, recon_hbm.at[r])
        return carry

    lax.fori_loop(0, _RPW, per_row, 0)


def kernel(x, W_enc, b_enc, W_dec, b_dec):
    x_cent = x - b_dec[None, :]

    pre_acts = pl.pallas_call(
        _enc_body,
        grid=(DICT_SIZE // _ENC_BD,),
        in_specs=[
            pl.BlockSpec((N_TOKENS, INPUT_DIM), lambda d: (0, 0)),
            pl.BlockSpec((_ENC_BD, INPUT_DIM), lambda d: (d, 0)),
            pl.BlockSpec((_ENC_BD,), lambda d: (d,)),
        ],
        out_specs=pl.BlockSpec((N_TOKENS, _ENC_BD), lambda d: (0, d)),
        out_shape=jax.ShapeDtypeStruct((N_TOKENS, DICT_SIZE), jnp.float32),
    )(x_cent, W_enc, b_enc)

    acts, bmp, thr = pl.pallas_call(
        _topk_body,
        grid=(N_TOKENS // _TOPK_BR,),
        in_specs=[pl.BlockSpec((_TOPK_BR, DICT_SIZE), lambda r: (r, 0))],
        out_specs=[
            pl.BlockSpec((_TOPK_BR, DICT_SIZE), lambda r: (r, 0)),
            pl.BlockSpec((_TOPK_BR, _NBLK), lambda r: (r, 0)),
            pl.BlockSpec((_TOPK_BR, _L), lambda r: (r, 0)),
        ],
        out_shape=[
            jax.ShapeDtypeStruct((N_TOKENS, DICT_SIZE), jnp.float32),
            jax.ShapeDtypeStruct((N_TOKENS, _NBLK), jnp.int32),
            jax.ShapeDtypeStruct((N_TOKENS, _L), jnp.float32),
        ],
    )(pre_acts)

    pa_blocks = pre_acts.reshape(N_TOKENS * _NBLK, _BLK)

    mesh = plsc.VectorSubcoreMesh(core_axis_name="c", subcore_axis_name="s",
                                  num_cores=_NC, num_subcores=_NS)
    recon = pl.kernel(
        _sc_body,
        out_type=jax.ShapeDtypeStruct((N_TOKENS, INPUT_DIM), jnp.float32),
        mesh=mesh,
        compiler_params=pltpu.CompilerParams(needs_layout_passes=False),
        scratch_types=[
            pltpu.VMEM((_NBLK,), jnp.int32),
            pltpu.VMEM((_L,), jnp.float32),
            pltpu.VMEM((K,), jnp.int32),
            pltpu.VMEM((K, _BLK), jnp.float32),
            pltpu.VMEM((K,), jnp.int32),
            pltpu.VMEM((K,), jnp.float32),
            pltpu.VMEM((4, 8, INPUT_DIM), jnp.float32),
            pltpu.VMEM((INPUT_DIM,), jnp.float32),
            pltpu.VMEM((INPUT_DIM,), jnp.float32),
            pltpu.SemaphoreType.DMA,
            pltpu.SemaphoreType.DMA,
            pltpu.SemaphoreType.DMA,
            pltpu.SemaphoreType.DMA,
        ],
    )(pa_blocks, bmp, thr, W_dec, b_dec)

    return (recon, acts)


# speedup vs baseline: 1.0896x; 1.0853x over previous
import functools

import jax
import jax.numpy as jnp
from jax import lax
from jax.experimental import pallas as pl
from jax.experimental.pallas import tpu as pltpu
from jax.experimental.pallas import tpu_sc as plsc

INPUT_DIM = 2048
DICT_SIZE = 32768
K = 64
N_TOKENS = 128

_ENC_BD = 2048
_TOPK_BR = 16
_BLK = 128
_NBLK = DICT_SIZE // _BLK

_NC = 2
_NS = 16
_NW = _NC * _NS
_RPW = N_TOKENS // _NW
_L = 16


def _enc_body(x_ref, w_ref, b_ref, out_ref):
    out_ref[...] = jax.lax.dot_general(
        x_ref[...], w_ref[...],
        (((1,), (1,)), ((), ())),
        preferred_element_type=jnp.float32,
    ) + b_ref[...][None, :]


def _topk_body(pa_ref, acts_ref, bmp_ref, thr_ref):
    v = pa_ref[...]
    bits = jax.lax.bitcast_convert_type(v, jnp.uint32)
    u = jnp.where(v >= 0.0, bits | jnp.uint32(0x80000000), ~bits)
    thr = jnp.zeros((v.shape[0], 1), jnp.uint32)
    for b in range(31, -1, -1):
        cand = thr | jnp.uint32(1 << b)
        cnt = jnp.sum((u >= cand).astype(jnp.int32), axis=1, keepdims=True)
        thr = jnp.where(cnt >= K, cand, thr)
    mask = u >= thr
    acts_ref[...] = jnp.where(mask, jnp.maximum(v, 0.0), 0.0)
    mi = mask.astype(jnp.int32).reshape(v.shape[0], _NBLK, _BLK)
    bmp_ref[...] = jnp.max(mi, axis=2)
    thr_f = jnp.where(
        thr >= jnp.uint32(0x80000000),
        jax.lax.bitcast_convert_type(thr & jnp.uint32(0x7FFFFFFF), jnp.float32),
        jax.lax.bitcast_convert_type(~thr, jnp.float32),
    )
    thr_ref[...] = jnp.broadcast_to(thr_f, (v.shape[0], _L))


def _sc_body(pab_hbm, bmp_hbm, thr_hbm, wdec_hbm, bdec_hbm, recon_hbm,
             bmv, thrv, blkids, cand, widx, wval, rowbuf, acc, bdec_v,
             sem0, sem1, sem2, sem3):
    wid = lax.axis_index("s") * _NC + lax.axis_index("c")
    rpw = bmp_hbm.shape[0] // _NW
    pltpu.sync_copy(bdec_hbm, bdec_v)
    iota = lax.iota(jnp.int32, _L)
    zi = jnp.zeros((_L,), jnp.int32)
    sems = (sem0, sem1, sem2, sem3)
    nchunk_bm = _NBLK // _L
    ngrp = K // _L

    def per_row(r0, carry):
        r = wid * rpw + r0
        pltpu.sync_copy(bmp_hbm.at[r], bmv)
        pltpu.sync_copy(thr_hbm.at[r], thrv)
        thr_s = jnp.broadcast_to(jnp.max(thrv[...], axis=0), (_L,))
        for j in range(K // _L):
            blkids[pl.ds(j * _L, _L)] = zi

        cnt = zi
        for c in range(nchunk_bm):
            m = bmv[pl.ds(c * _L, _L)] != 0
            pos = jnp.clip(cnt + plsc.cumsum(m.astype(jnp.int32)) - 1, 0, K - 1)
            plsc.store_scatter(blkids, [pos], c * _L + iota, mask=m)
            cnt = cnt + plsc.all_reduce_population_count(m)
        nblk = jnp.max(cnt, axis=0)

        base = jnp.broadcast_to(r * _NBLK, (_L,))
        cps = []
        for gch in range(K // _L):
            idxv = base + blkids[pl.ds(gch * _L, _L)]
            cps.append(pltpu.async_copy(
                pab_hbm.at[idxv], cand.at[pl.ds(gch * _L, _L)], sems[gch]))
        for cp in cps:
            cp.wait()

        zf = jnp.zeros((_L,), jnp.float32)
        for j in range(K // _L):
            widx[pl.ds(j * _L, _L)] = zi
            wval[pl.ds(j * _L, _L)] = zf

        def sel_block(b, wcnt):
            bch = b // _L
            bb = bch * _L
            blk_chunk = blkids[pl.ds(bb, _L)]
            blk_s = jnp.max(jnp.where(iota == b - bb, blk_chunk, 0), axis=0)
            colbase = jnp.broadcast_to(blk_s * _BLK, (_L,))
            for o in range(_BLK // _L):
                v = cand[b, pl.ds(o * _L, _L)]
                m = v >= thr_s
                pos = jnp.clip(
                    wcnt + plsc.cumsum(m.astype(jnp.int32)) - 1, 0, K - 1)
                plsc.store_scatter(widx, [pos], colbase + o * _L + iota, mask=m)
                plsc.store_scatter(wval, [pos], jnp.maximum(v, 0.0), mask=m)
                wcnt = wcnt + plsc.all_reduce_population_count(m)
            return wcnt

        lax.fori_loop(0, nblk, sel_block, zi)

        _GR = 8
        _NG = K // _GR
        _NB = 4
        _UNR = 4

        cps = {}
        for g in range(3):
            cps[g] = pltpu.async_copy(
                wdec_hbm.at[widx.at[pl.ds(g * _GR, _GR)]],
                rowbuf.at[g % _NB], sems[g % _NB])

        def init_body(i, _):
            for oo in range(_UNR):
                s = pl.ds((i * _UNR + oo) * _L, _L)
                acc[s] = bdec_v[s]
            return 0

        lax.fori_loop(0, INPUT_DIM // _L // _UNR, init_body, 0)
        for g in range(_NG):
            cps[g].wait()
            if g + 3 < _NG:
                cps[g + 3] = pltpu.async_copy(
                    wdec_hbm.at[widx.at[pl.ds((g + 3) * _GR, _GR)]],
                    rowbuf.at[(g + 3) % _NB], sems[(g + 3) % _NB])
            vchunk = wval[pl.ds((g // 2) * _L, _L)]
            half = (g % 2) * _GR
            vals = [jnp.broadcast_to(
                        jnp.sum(jnp.where(iota == half + j, vchunk, 0.0),
                                axis=0), (_L,))
                    for j in range(_GR)]

            def acc_body(i, _, g=g, vals=vals):
                for oo in range(_UNR):
                    s = pl.ds((i * _UNR + oo) * _L, _L)
                    a = acc[s]
                    for j in range(_GR):
                        a = a + vals[j] * rowbuf[g % _NB, j, s]
                    acc[s] = a
                return 0

            lax.fori_loop(0, INPUT_DIM // _L // _UNR, acc_body, 0)
        pltpu.sync_copy(acc, recon_hbm.at[r])
        return carry

    lax.fori_loop(0, rpw, per_row, 0)


def kernel(x, W_enc, b_enc, W_dec, b_dec):
    x_cent = x - b_dec[None, :]

    pre_acts = pl.pallas_call(
        _enc_body,
        grid=(DICT_SIZE // _ENC_BD,),
        in_specs=[
            pl.BlockSpec((N_TOKENS, INPUT_DIM), lambda d: (0, 0)),
            pl.BlockSpec((_ENC_BD, INPUT_DIM), lambda d: (d, 0)),
            pl.BlockSpec((_ENC_BD,), lambda d: (d,)),
        ],
        out_specs=pl.BlockSpec((N_TOKENS, _ENC_BD), lambda d: (0, d)),
        out_shape=jax.ShapeDtypeStruct((N_TOKENS, DICT_SIZE), jnp.float32),
    )(x_cent, W_enc, b_enc)

    _S = 2
    nt_h = N_TOKENS // _S
    mesh = plsc.VectorSubcoreMesh(core_axis_name="c", subcore_axis_name="s",
                                  num_cores=_NC, num_subcores=_NS)

    acts_parts, recon_parts = [], []
    for h in range(_S):
        pa_h = jax.lax.slice_in_dim(pre_acts, h * nt_h, (h + 1) * nt_h, axis=0)
        acts_h, bmp_h, thr_h = pl.pallas_call(
            _topk_body,
            grid=(nt_h // _TOPK_BR,),
            in_specs=[pl.BlockSpec((_TOPK_BR, DICT_SIZE), lambda r: (r, 0))],
            out_specs=[
                pl.BlockSpec((_TOPK_BR, DICT_SIZE), lambda r: (r, 0)),
                pl.BlockSpec((_TOPK_BR, _NBLK), lambda r: (r, 0)),
                pl.BlockSpec((_TOPK_BR, _L), lambda r: (r, 0)),
            ],
            out_shape=[
                jax.ShapeDtypeStruct((nt_h, DICT_SIZE), jnp.float32),
                jax.ShapeDtypeStruct((nt_h, _NBLK), jnp.int32),
                jax.ShapeDtypeStruct((nt_h, _L), jnp.float32),
            ],
        )(pa_h)

        pa_blocks_h = pa_h.reshape(nt_h * _NBLK, _BLK)
        recon_h = pl.kernel(
            _sc_body,
            out_type=jax.ShapeDtypeStruct((nt_h, INPUT_DIM), jnp.float32),
            mesh=mesh,
            compiler_params=pltpu.CompilerParams(needs_layout_passes=False),
            scratch_types=[
                pltpu.VMEM((_NBLK,), jnp.int32),
                pltpu.VMEM((_L,), jnp.float32),
                pltpu.VMEM((K,), jnp.int32),
                pltpu.VMEM((K, _BLK), jnp.float32),
                pltpu.VMEM((K,), jnp.int32),
                pltpu.VMEM((K,), jnp.float32),
                pltpu.VMEM((4, 8, INPUT_DIM), jnp.float32),
                pltpu.VMEM((INPUT_DIM,), jnp.float32),
                pltpu.VMEM((INPUT_DIM,), jnp.float32),
                pltpu.SemaphoreType.DMA,
                pltpu.SemaphoreType.DMA,
                pltpu.SemaphoreType.DMA,
                pltpu.SemaphoreType.DMA,
            ],
        )(pa_blocks_h, bmp_h, thr_h, W_dec, b_dec)
        acts_parts.append(acts_h)
        recon_parts.append(recon_h)

    acts = jnp.concatenate(acts_parts, axis=0)
    recon = jnp.concatenate(recon_parts, axis=0)
    return (recon, acts)


# speedup vs baseline: 1.1061x; 1.0152x over previous
import functools

import jax
import jax.numpy as jnp
from jax import lax
from jax.experimental import pallas as pl
from jax.experimental.pallas import tpu as pltpu
from jax.experimental.pallas import tpu_sc as plsc

INPUT_DIM = 2048
DICT_SIZE = 32768
K = 64
N_TOKENS = 128

_ENC_BD = 2048
_TOPK_BR = 16
_BLK = 128
_NBLK = DICT_SIZE // _BLK

_NC = 2
_NS = 16
_NW = _NC * _NS
_L = 16

_S = 2
_NT_H = N_TOKENS // _S
_RPW = _NT_H // _NW

_GR = 8
_NGR = K // _GR
_NB = 4
_AHEAD = 3


def _enc_body(x_ref, w_ref, b_ref, out_ref):
    out_ref[...] = jax.lax.dot_general(
        x_ref[...], w_ref[...],
        (((1,), (1,)), ((), ())),
        preferred_element_type=jnp.float32,
    ) + b_ref[...][None, :]


def _topk_body(pa_ref, acts_ref, bmp_ref, thr_ref):
    v = pa_ref[...]
    bits = jax.lax.bitcast_convert_type(v, jnp.uint32)
    u = jnp.where(v >= 0.0, bits | jnp.uint32(0x80000000), ~bits)
    thr = jnp.zeros((v.shape[0], 1), jnp.uint32)
    for b in range(31, -1, -1):
        cand = thr | jnp.uint32(1 << b)
        cnt = jnp.sum((u >= cand).astype(jnp.int32), axis=1, keepdims=True)
        thr = jnp.where(cnt >= K, cand, thr)
    mask = u >= thr
    acts_ref[...] = jnp.where(mask, jnp.maximum(v, 0.0), 0.0)
    mi = mask.astype(jnp.int32).reshape(v.shape[0], _NBLK, _BLK)
    bmp_ref[...] = jnp.max(mi, axis=2)
    thr_f = jnp.where(
        thr >= jnp.uint32(0x80000000),
        jax.lax.bitcast_convert_type(thr & jnp.uint32(0x7FFFFFFF), jnp.float32),
        jax.lax.bitcast_convert_type(~thr, jnp.float32),
    )
    thr_ref[...] = jnp.broadcast_to(thr_f, (v.shape[0], _L))


def _sc_body(pab_hbm, bmp_hbm, thr_hbm, wdec_hbm, bdec_hbm, recon_hbm,
             bmv_all, thr_all, blkids_all, cand2, widx_all, wval_all,
             rowbuf, acc_all, bdec_v,
             semc0, semc1, semd0, semd1, semd2, semd3, semo):
    wid = lax.axis_index("s") * _NC + lax.axis_index("c")
    r0 = wid * _RPW
    iota = lax.iota(jnp.int32, _L)
    zi = jnp.zeros((_L,), jnp.int32)
    zf = jnp.zeros((_L,), jnp.float32)
    csems = (semc0, semc1)
    dsems = (semd0, semd1, semd2, semd3)

    cp_bm = pltpu.async_copy(bmp_hbm.at[pl.ds(r0, _RPW)], bmv_all, semd0)
    cp_th = pltpu.async_copy(thr_hbm.at[pl.ds(r0, _RPW)], thr_all, semd1)
    pltpu.sync_copy(bdec_hbm, bdec_v)
    cp_bm.wait()
    cp_th.wait()

    def compact_and_fire(r):
        for j in range(K // _L):
            blkids_all[r, pl.ds(j * _L, _L)] = zi
        cnt = zi
        for c in range(_NBLK // _L):
            m = bmv_all[r, pl.ds(c * _L, _L)] != 0
            pos = jnp.clip(cnt + plsc.cumsum(m.astype(jnp.int32)) - 1, 0, K - 1)
            plsc.store_scatter(blkids_all.at[r], [pos], c * _L + iota, mask=m)
            cnt = cnt + plsc.all_reduce_population_count(m)
        base = jnp.broadcast_to((r0 + r) * _NBLK, (_L,))
        cps = []
        for gch in range(K // _L):
            idxv = base + blkids_all[r, pl.ds(gch * _L, _L)]
            cps.append(pltpu.async_copy(
                pab_hbm.at[idxv], cand2.at[r % 2].at[pl.ds(gch * _L, _L)],
                csems[r % 2]))
        return cnt, cps

    cnts = {}
    cnts[0], cps = compact_and_fire(0)
    for r in range(_RPW):
        if r + 1 < _RPW:
            cnts[r + 1], cps_next = compact_and_fire(r + 1)
        for cp in cps:
            cp.wait()
        if r + 1 < _RPW:
            cps = cps_next
        nblk = jnp.max(cnts[r], axis=0)
        thr_s = jnp.broadcast_to(jnp.max(thr_all[r], axis=0), (_L,))
        for j in range(K // _L):
            widx_all[r, pl.ds(j * _L, _L)] = zi
            wval_all[r, pl.ds(j * _L, _L)] = zf

        def sel_block(b, wcnt, r=r, thr_s=thr_s):
            bch = b // _L
            bb = bch * _L
            blk_chunk = blkids_all[r, pl.ds(bb, _L)]
            blk_s = jnp.max(jnp.where(iota == b - bb, blk_chunk, 0), axis=0)
            colbase = jnp.broadcast_to(blk_s * _BLK, (_L,))
            for o in range(_BLK // _L):
                v = cand2[r % 2, b, pl.ds(o * _L, _L)]
                m = v >= thr_s
                pos = jnp.clip(
                    wcnt + plsc.cumsum(m.astype(jnp.int32)) - 1, 0, K - 1)
                plsc.store_scatter(widx_all.at[r], [pos],
                                   colbase + o * _L + iota, mask=m)
                plsc.store_scatter(wval_all.at[r], [pos],
                                   jnp.maximum(v, 0.0), mask=m)
                wcnt = wcnt + plsc.all_reduce_population_count(m)
            return wcnt

        lax.fori_loop(0, nblk, sel_block, zi)

    _NGT = _RPW * _NGR

    def fire(G):
        return pltpu.async_copy(
            wdec_hbm.at[widx_all.at[G // _NGR].at[pl.ds((G % _NGR) * _GR, _GR)]],
            rowbuf.at[G % _NB], dsems[G % _NB])

    cpd = {}
    for G in range(min(_AHEAD, _NGT)):
        cpd[G] = fire(G)

    _UNR = 4
    ocps = []
    for r in range(_RPW):
        def init_body(i, _, r=r):
            for oo in range(_UNR):
                s = pl.ds((i * _UNR + oo) * _L, _L)
                acc_all[r, s] = bdec_v[s]
            return 0

        lax.fori_loop(0, INPUT_DIM // _L // _UNR, init_body, 0)
        for g in range(_NGR):
            G = r * _NGR + g
            cpd[G].wait()
            if G + _AHEAD < _NGT:
                cpd[G + _AHEAD] = fire(G + _AHEAD)
            vchunk = wval_all[r, pl.ds((g // 2) * _L, _L)]
            half = (g % 2) * _GR
            vals = [jnp.broadcast_to(
                        jnp.sum(jnp.where(iota == half + j, vchunk, 0.0),
                                axis=0), (_L,))
                    for j in range(_GR)]

            def acc_body(i, _, r=r, G=G, vals=vals):
                for oo in range(_UNR):
                    s = pl.ds((i * _UNR + oo) * _L, _L)
                    a = acc_all[r, s]
                    for j in range(_GR):
                        a = a + vals[j] * rowbuf[G % _NB, j, s]
                    acc_all[r, s] = a
                return 0

            lax.fori_loop(0, INPUT_DIM // _L // _UNR, acc_body, 0)
        ocps.append(pltpu.async_copy(acc_all.at[r], recon_hbm.at[r0 + r], semo))
    for cp in ocps:
        cp.wait()


def kernel(x, W_enc, b_enc, W_dec, b_dec):
    x_cent = x - b_dec[None, :]

    pre_acts = pl.pallas_call(
        _enc_body,
        grid=(DICT_SIZE // _ENC_BD,),
        in_specs=[
            pl.BlockSpec((N_TOKENS, INPUT_DIM), lambda d: (0, 0)),
            pl.BlockSpec((_ENC_BD, INPUT_DIM), lambda d: (d, 0)),
            pl.BlockSpec((_ENC_BD,), lambda d: (d,)),
        ],
        out_specs=pl.BlockSpec((N_TOKENS, _ENC_BD), lambda d: (0, d)),
        out_shape=jax.ShapeDtypeStruct((N_TOKENS, DICT_SIZE), jnp.float32),
    )(x_cent, W_enc, b_enc)

    mesh = plsc.VectorSubcoreMesh(core_axis_name="c", subcore_axis_name="s",
                                  num_cores=_NC, num_subcores=_NS)

    acts_parts, recon_parts = [], []
    for h in range(_S):
        pa_h = jax.lax.slice_in_dim(pre_acts, h * _NT_H, (h + 1) * _NT_H,
                                    axis=0)
        acts_h, bmp_h, thr_h = pl.pallas_call(
            _topk_body,
            grid=(_NT_H // _TOPK_BR,),
            in_specs=[pl.BlockSpec((_TOPK_BR, DICT_SIZE), lambda r: (r, 0))],
            out_specs=[
                pl.BlockSpec((_TOPK_BR, DICT_SIZE), lambda r: (r, 0)),
                pl.BlockSpec((_TOPK_BR, _NBLK), lambda r: (r, 0)),
                pl.BlockSpec((_TOPK_BR, _L), lambda r: (r, 0)),
            ],
            out_shape=[
                jax.ShapeDtypeStruct((_NT_H, DICT_SIZE), jnp.float32),
                jax.ShapeDtypeStruct((_NT_H, _NBLK), jnp.int32),
                jax.ShapeDtypeStruct((_NT_H, _L), jnp.float32),
            ],
        )(pa_h)

        pa_blocks_h = pa_h.reshape(_NT_H * _NBLK, _BLK)
        recon_h = pl.kernel(
            _sc_body,
            out_type=jax.ShapeDtypeStruct((_NT_H, INPUT_DIM), jnp.float32),
            mesh=mesh,
            compiler_params=pltpu.CompilerParams(needs_layout_passes=False),
            scratch_types=[
                pltpu.VMEM((_RPW, _NBLK), jnp.int32),
                pltpu.VMEM((_RPW, _L), jnp.float32),
                pltpu.VMEM((_RPW, K), jnp.int32),
                pltpu.VMEM((2, K, _BLK), jnp.float32),
                pltpu.VMEM((_RPW, K), jnp.int32),
                pltpu.VMEM((_RPW, K), jnp.float32),
                pltpu.VMEM((_NB, _GR, INPUT_DIM), jnp.float32),
                pltpu.VMEM((_RPW, INPUT_DIM), jnp.float32),
                pltpu.VMEM((INPUT_DIM,), jnp.float32),
                pltpu.SemaphoreType.DMA,
                pltpu.SemaphoreType.DMA,
                pltpu.SemaphoreType.DMA,
                pltpu.SemaphoreType.DMA,
                pltpu.SemaphoreType.DMA,
                pltpu.SemaphoreType.DMA,
                pltpu.SemaphoreType.DMA,
            ],
        )(pa_blocks_h, bmp_h, thr_h, W_dec, b_dec)
        acts_parts.append(acts_h)
        recon_parts.append(recon_h)

    acts = jnp.concatenate(acts_parts, axis=0)
    recon = jnp.concatenate(recon_parts, axis=0)
    return (recon, acts)
